# baseline (speedup 1.0000x reference)
import functools

import jax
import jax.numpy as jnp
from jax import lax
from jax.experimental import pallas as pl
from jax.experimental.pallas import tpu as pltpu
from jax.experimental.pallas import tpu_sc as plsc

D = 16
NC = 2
NS = 16
NW = NC * NS
CHUNK = 1664


@functools.lru_cache(maxsize=None)
def _make_gather(n_rows: int):
    assert n_rows % NW == 0
    b_per_w = n_rows // NW
    assert b_per_w % CHUNK == 0
    n_chunk = b_per_w // CHUNK
    mesh = plsc.VectorSubcoreMesh(core_axis_name="c", subcore_axis_name="s")

    @functools.partial(
        pl.kernel,
        out_type=jax.ShapeDtypeStruct((n_rows, D), jnp.float32),
        mesh=mesh,
        compiler_params=pltpu.CompilerParams(use_tc_tiling_on_sc=False),
        scratch_types=[
            pltpu.VMEM((n_chunk, CHUNK), jnp.int32),
            pltpu.VMEM((2, CHUNK, D), jnp.float32),
            pltpu.SemaphoreType.DMA,
            pltpu.SemaphoreType.DMA,
        ],
    )
    def gather_kernel(idx_hbm, table_hbm, out_hbm, idx_v, rows_v, gsem, ssem):
        wid = lax.axis_index("s") * NC + lax.axis_index("c")
        base = wid * b_per_w
        pltpu.sync_copy(idx_hbm.at[wid], idx_v)
        gathers = [
            pltpu.make_async_copy(
                table_hbm.at[idx_v.at[c]], rows_v.at[c % 2], gsem
            )
            for c in range(n_chunk)
        ]
        stores = [
            pltpu.make_async_copy(
                rows_v.at[c % 2],
                out_hbm.at[pl.ds(base + c * CHUNK, CHUNK)],
                ssem,
            )
            for c in range(n_chunk)
        ]
        gathers[0].start()
        for c in range(n_chunk):
            gathers[c].wait()
            if c + 1 < n_chunk:
                gathers[c + 1].start()
            stores[c].start()
            stores[c].wait()

    return gather_kernel


def kernel(x, table):
    b, s = x.shape
    n_rows = b * s
    b_per_w = n_rows // NW
    n_chunk = b_per_w // CHUNK
    idx = x.reshape(NW, n_chunk, CHUNK).astype(jnp.int32)
    out = _make_gather(n_rows)(idx, table)
    return out.reshape(b, s, D)


# speedup vs baseline: 1.0048x; 1.0048x over previous
import functools

import jax
import jax.numpy as jnp
from jax import lax
from jax.experimental import pallas as pl
from jax.experimental.pallas import tpu as pltpu
from jax.experimental.pallas import tpu_sc as plsc

D = 16
NC = 2
NS = 16
NW = NC * NS
CHUNK = 1664
NBUF = 4


@functools.lru_cache(maxsize=None)
def _make_gather(n_rows: int):
    assert n_rows % NW == 0
    b_per_w = n_rows // NW
    assert b_per_w % CHUNK == 0
    n_chunk = b_per_w // CHUNK
    mesh = plsc.VectorSubcoreMesh(core_axis_name="c", subcore_axis_name="s")

    @functools.partial(
        pl.kernel,
        out_type=jax.ShapeDtypeStruct((n_rows, D), jnp.float32),
        mesh=mesh,
        compiler_params=pltpu.CompilerParams(use_tc_tiling_on_sc=False),
        scratch_types=[
            pltpu.VMEM((n_chunk, CHUNK), jnp.int32),
            pltpu.VMEM((NBUF, CHUNK, D), jnp.float32),
            pltpu.SemaphoreType.DMA((NBUF,)),
            pltpu.SemaphoreType.DMA((NBUF,)),
        ],
    )
    def gather_kernel(idx_hbm, table_hbm, out_hbm, idx_v, rows_v, gsem, ssem):
        wid = lax.axis_index("s") * NC + lax.axis_index("c")
        base = wid * b_per_w
        pltpu.sync_copy(idx_hbm.at[wid], idx_v)
        gathers = [
            pltpu.make_async_copy(
                table_hbm.at[idx_v.at[c]],
                rows_v.at[c % NBUF],
                gsem.at[c % NBUF],
            )
            for c in range(n_chunk)
        ]
        stores = [
            pltpu.make_async_copy(
                rows_v.at[c % NBUF],
                out_hbm.at[pl.ds(base + c * CHUNK, CHUNK)],
                ssem.at[c % NBUF],
            )
            for c in range(n_chunk)
        ]
        for c in range(min(NBUF, n_chunk)):
            gathers[c].start()
        for c in range(n_chunk):
            p = c - 1
            if p >= 0 and p + NBUF < n_chunk:
                stores[p].wait()
                gathers[p + NBUF].start()
            gathers[c].wait()
            stores[c].start()
        for c in range(max(0, n_chunk - NBUF), n_chunk):
            stores[c].wait()

    return gather_kernel


def kernel(x, table):
    b, s = x.shape
    n_rows = b * s
    b_per_w = n_rows // NW
    n_chunk = b_per_w // CHUNK
    idx = x.reshape(NW, n_chunk, CHUNK).astype(jnp.int32)
    out = _make_gather(n_rows)(idx, table)
    return out.reshape(b, s, D)


# speedup vs baseline: 1.6174x; 1.6097x over previous
import functools

import jax
import jax.numpy as jnp
from jax import lax
from jax.experimental import pallas as pl
from jax.experimental.pallas import tpu as pltpu
from jax.experimental.pallas import tpu_sc as plsc

D = 16
NC = 2
NS = 16
NW = NC * NS
NBUF = 2


@functools.lru_cache(maxsize=None)
def _make_gather(batch: int, n_col: int, n_cat: int):
    assert batch % (NW * 128) == 0
    b_per_w = batch // NW
    blk = 8 * b_per_w
    per_j = D * batch
    mesh = plsc.VectorSubcoreMesh(core_axis_name="c", subcore_axis_name="s")

    @functools.partial(
        pl.kernel,
        out_type=jax.ShapeDtypeStruct((n_col * per_j,), jnp.float32),
        mesh=mesh,
        compiler_params=pltpu.CompilerParams(
            use_tc_tiling_on_sc=False, needs_layout_passes=False
        ),
        scratch_types=[
            pltpu.VMEM((n_col, b_per_w), jnp.int32),
            pltpu.VMEM((NBUF, b_per_w, D), jnp.float32),
            pltpu.VMEM((NBUF, 2 * blk), jnp.float32),
            pltpu.SemaphoreType.DMA((NBUF,)),
            pltpu.SemaphoreType.DMA((NBUF,)),
        ],
    )
    def gather_kernel(idx_hbm, table_hbm, out_hbm, idx_v, rows_v, outt_v,
                      gsem, ssem):
        wid = lax.axis_index("s") * NC + lax.axis_index("c")
        base = wid * b_per_w
        pltpu.sync_copy(idx_hbm.at[:, pl.ds(base, b_per_w)], idx_v)
        gathers = [
            pltpu.make_async_copy(
                table_hbm.at[idx_v.at[j]],
                rows_v.at[j % NBUF],
                gsem.at[j % NBUF],
            )
            for j in range(n_col)
        ]
        stores = [
            [
                pltpu.make_async_copy(
                    outt_v.at[j % NBUF, pl.ds(rt * blk, blk)],
                    out_hbm.at[
                        pl.ds(j * per_j + rt * (8 * batch) + wid * blk, blk)
                    ],
                    ssem.at[j % NBUF],
                )
                for rt in range(2)
            ]
            for j in range(n_col)
        ]
        feat = lax.iota(jnp.int32, D)
        foff = (feat // 8) * blk + (feat % 8) * 128

        def transpose_block(buf):
            def body(c, _):
                coff = (c // 128) * 1024 + (c % 128)
                row = plsc.load_gather(
                    rows_v.at[buf], [jnp.full((D,), c, jnp.int32), feat]
                )
                plsc.store_scatter(outt_v.at[buf], [foff + coff], row)
                return 0

            lax.fori_loop(0, b_per_w, body, 0, unroll=8)

        gathers[0].start()
        for j in range(n_col):
            bj = j % NBUF
            gathers[j].wait()
            if j + 1 < n_col:
                gathers[j + 1].start()
            if j >= NBUF:
                for s in stores[j - NBUF]:
                    s.wait()
            transpose_block(bj)
            for s in stores[j]:
                s.start()
        for j in range(max(0, n_col - NBUF), n_col):
            for s in stores[j]:
                s.wait()

    return gather_kernel


def kernel(x, table):
    b, s = x.shape
    n_cat, d = table.shape
    xt = x.T.astype(jnp.int32)
    flat = _make_gather(b, s, n_cat)(xt, table)
    out = flat.reshape(s, d // 8, b // 128, 8, 128)
    return out.transpose(2, 4, 0, 1, 3).reshape(b, s, d)


# speedup vs baseline: 1.6644x; 1.0290x over previous
import functools

import jax
import jax.numpy as jnp
from jax import lax
from jax.experimental import pallas as pl
from jax.experimental.pallas import tpu as pltpu
from jax.experimental.pallas import tpu_sc as plsc

D = 16
NC = 2
NS = 16
NW = NC * NS
NBUF = 2


DW = 1024
DNF = 30


@functools.lru_cache(maxsize=None)
def _make_detile(n_cat: int):
    n_full = n_cat // DW
    n_extra = n_full - DNF * NW
    rem = n_cat % 128
    tail = n_cat - n_full * DW - rem
    mesh = plsc.VectorSubcoreMesh(core_axis_name="c", subcore_axis_name="s")

    @functools.partial(
        pl.kernel,
        out_type=jax.ShapeDtypeStruct((n_cat * D,), jnp.float32),
        mesh=mesh,
        compiler_params=pltpu.CompilerParams(needs_layout_passes=False),
        scratch_types=[
            pltpu.VMEM((NBUF, D, DW), jnp.float32),
            pltpu.VMEM((NBUF, DW * D), jnp.float32),
            pltpu.SemaphoreType.DMA((NBUF,)),
            pltpu.SemaphoreType.DMA((NBUF,)),
        ],
    )
    def detile_kernel(tt_hbm, rem_hbm, out_hbm, in_v, out_v, gsem, ssem):
        wid = lax.axis_index("s") * NC + lax.axis_index("c")
        lo = DNF * wid + jnp.minimum(wid, n_extra)
        feat = lax.iota(jnp.int32, D)

        def cat0(k):
            return pl.multiple_of((lo + k) * DW, 128)

        def transpose_chunk(buf, width):
            def body(c, _):
                col = plsc.load_gather(
                    in_v.at[buf], [feat, jnp.full((D,), c, jnp.int32)]
                )
                out_v[buf, pl.ds(c * D, D)] = col
                return 0

            lax.fori_loop(0, width, body, 0, unroll=8)

        def load_chunk(buf, c0, width, sem):
            return pltpu.make_async_copy(
                tt_hbm.at[:, pl.ds(c0, width)],
                in_v.at[buf, :, pl.ds(0, width)],
                sem,
            )

        def store_chunk(buf, c0, width, sem):
            return pltpu.make_async_copy(
                out_v.at[buf, pl.ds(0, width * D)],
                out_hbm.at[pl.ds(c0 * D, width * D)],
                sem,
            )

        loads = [load_chunk(k % NBUF, cat0(k), DW, gsem.at[k % NBUF])
                 for k in range(DNF)]
        stores = [store_chunk(k % NBUF, cat0(k), DW, ssem.at[k % NBUF])
                  for k in range(DNF)]
        loads[0].start()
        for k in range(DNF):
            bk = k % NBUF
            loads[k].wait()
            if k + 1 < DNF:
                loads[k + 1].start()
            if k >= NBUF:
                stores[k - NBUF].wait()
            transpose_chunk(bk, DW)
            stores[k].start()
        for k in range(max(0, DNF - NBUF), DNF):
            stores[k].wait()

        @pl.when(wid < n_extra)
        def _extra():
            c0 = cat0(DNF)
            load_chunk(0, c0, DW, gsem.at[0]).start()
            load_chunk(0, c0, DW, gsem.at[0]).wait()
            transpose_chunk(0, DW)
            store_chunk(0, c0, DW, ssem.at[0]).start()
            store_chunk(0, c0, DW, ssem.at[0]).wait()

        @pl.when(wid == NW - 1)
        def _tail():
            t0 = n_full * DW
            load_chunk(1, t0, tail, gsem.at[1]).start()
            load_chunk(1, t0, tail, gsem.at[1]).wait()
            transpose_chunk(1, tail)
            store_chunk(1, t0, tail, ssem.at[1]).start()
            store_chunk(1, t0, tail, ssem.at[1]).wait()

        @pl.when(wid == 0)
        def _rem():
            r0 = (n_cat - rem) * D
            pltpu.sync_copy(rem_hbm, out_v.at[0, pl.ds(0, rem * D)])
            pltpu.sync_copy(
                out_v.at[0, pl.ds(0, rem * D)], out_hbm.at[pl.ds(r0, rem * D)]
            )

    return detile_kernel


@functools.lru_cache(maxsize=None)
def _make_gather(batch: int, n_col: int, n_cat: int):
    assert batch % (NW * 128) == 0
    b_per_w = batch // NW
    blk = 8 * b_per_w
    per_j = D * batch
    mesh = plsc.VectorSubcoreMesh(core_axis_name="c", subcore_axis_name="s")

    @functools.partial(
        pl.kernel,
        out_type=jax.ShapeDtypeStruct((n_col * per_j,), jnp.float32),
        mesh=mesh,
        compiler_params=pltpu.CompilerParams(
            use_tc_tiling_on_sc=False, needs_layout_passes=False
        ),
        scratch_types=[
            pltpu.VMEM((n_col, b_per_w), jnp.int32),
            pltpu.VMEM((NBUF, b_per_w, D), jnp.float32),
            pltpu.VMEM((NBUF, 2 * blk), jnp.float32),
            pltpu.SemaphoreType.DMA((NBUF,)),
            pltpu.SemaphoreType.DMA((NBUF,)),
        ],
    )
    def gather_kernel(idx_hbm, table_hbm, out_hbm, idx_v, rows_v, outt_v,
                      gsem, ssem):
        wid = lax.axis_index("s") * NC + lax.axis_index("c")
        base = wid * b_per_w
        pltpu.sync_copy(idx_hbm.at[:, pl.ds(base, b_per_w)], idx_v)
        gathers = [
            pltpu.make_async_copy(
                table_hbm.at[idx_v.at[j]],
                rows_v.at[j % NBUF],
                gsem.at[j % NBUF],
            )
            for j in range(n_col)
        ]
        stores = [
            [
                pltpu.make_async_copy(
                    outt_v.at[j % NBUF, pl.ds(rt * blk, blk)],
                    out_hbm.at[
                        pl.ds(j * per_j + rt * (8 * batch) + wid * blk, blk)
                    ],
                    ssem.at[j % NBUF],
                )
                for rt in range(2)
            ]
            for j in range(n_col)
        ]
        feat = lax.iota(jnp.int32, D)
        foff = (feat // 8) * blk + (feat % 8) * 128

        def transpose_block(buf):
            def body(c, _):
                coff = (c // 128) * 1024 + (c % 128)
                row = plsc.load_gather(
                    rows_v.at[buf], [jnp.full((D,), c, jnp.int32), feat]
                )
                plsc.store_scatter(outt_v.at[buf], [foff + coff], row)
                return 0

            lax.fori_loop(0, b_per_w, body, 0, unroll=8)

        gathers[0].start()
        for j in range(n_col):
            bj = j % NBUF
            gathers[j].wait()
            if j + 1 < n_col:
                gathers[j + 1].start()
            if j >= NBUF:
                for s in stores[j - NBUF]:
                    s.wait()
            transpose_block(bj)
            for s in stores[j]:
                s.start()
        for j in range(max(0, n_col - NBUF), n_col):
            for s in stores[j]:
                s.wait()

    return gather_kernel


def kernel(x, table):
    b, s = x.shape
    n_cat, d = table.shape
    xt = x.T.astype(jnp.int32)
    rem = n_cat % 128
    rem_rows = table[n_cat - rem:, :].reshape(-1)
    table_lin = _make_detile(n_cat)(table.T, rem_rows).reshape(n_cat, d)
    flat = _make_gather(b, s, n_cat)(xt, table_lin)
    out = flat.reshape(s, d // 8, b // 128, 8, 128)
    return out.transpose(2, 4, 0, 1, 3).reshape(b, s, d)


# speedup vs baseline: 1.7069x; 1.0256x over previous
import functools

import jax
import jax.numpy as jnp
from jax import lax
from jax.experimental import pallas as pl
from jax.experimental.pallas import tpu as pltpu
from jax.experimental.pallas import tpu_sc as plsc

D = 16
NC = 2
NS = 16
NW = NC * NS
NBUF = 2


DW = 1024
DNF = 30


@functools.lru_cache(maxsize=None)
def _make_detile(n_cat: int):
    n_full = n_cat // DW
    n_extra = n_full - DNF * NW
    rem = n_cat % 128
    tail = n_cat - n_full * DW - rem
    mesh = plsc.VectorSubcoreMesh(core_axis_name="c", subcore_axis_name="s")

    @functools.partial(
        pl.kernel,
        out_type=jax.ShapeDtypeStruct((n_cat * D,), jnp.float32),
        mesh=mesh,
        compiler_params=pltpu.CompilerParams(needs_layout_passes=False),
        scratch_types=[
            pltpu.VMEM((D * DW,), jnp.float32),
            pltpu.VMEM((D * DW,), jnp.float32),
            pltpu.VMEM((D * DW,), jnp.float32),
            pltpu.VMEM((D * DW,), jnp.float32),
            pltpu.SemaphoreType.DMA((NBUF,)),
            pltpu.SemaphoreType.DMA((NBUF,)),
        ],
    )
    def detile_kernel(tt_hbm, rem_hbm, out_hbm, in0, in1, ob0, ob1,
                      gsem, ssem):
        wid = lax.axis_index("s") * NC + lax.axis_index("c")
        lo = DNF * wid + jnp.minimum(wid, n_extra)
        feat = lax.iota(jnp.int32, D)
        inb = [in0, in1]
        outb = [ob0, ob1]

        def cat0(k):
            return pl.multiple_of((lo + k) * DW, 128)

        def transpose_chunk(buf, width):
            src, dst = inb[buf], outb[buf]

            def body(c, lvec):
                col = plsc.load_gather(src, [lvec])
                dst[pl.ds(c * D, D)] = col
                return lvec + 1

            lax.fori_loop(0, width, body, feat * DW, unroll=8)

        def load_chunk(buf, c0, width):
            return [
                pltpu.make_async_copy(
                    tt_hbm.at[f, pl.ds(c0, width)],
                    inb[buf].at[pl.ds(f * DW, width)],
                    gsem.at[buf],
                )
                for f in range(D)
            ]

        def store_chunk(buf, c0, width):
            return pltpu.make_async_copy(
                outb[buf].at[pl.ds(0, width * D)],
                out_hbm.at[pl.ds(c0 * D, width * D)],
                ssem.at[buf],
            )

        def run_chunk_sync(buf, c0, width):
            for c in load_chunk(buf, c0, width):
                c.start()
            for c in load_chunk(buf, c0, width):
                c.wait()
            transpose_chunk(buf, width)
            store_chunk(buf, c0, width).start()
            store_chunk(buf, c0, width).wait()

        loads = [load_chunk(k % NBUF, cat0(k), DW) for k in range(DNF)]
        stores = [store_chunk(k % NBUF, cat0(k), DW) for k in range(DNF)]
        for c in loads[0]:
            c.start()
        for k in range(DNF):
            bk = k % NBUF
            for c in loads[k]:
                c.wait()
            if k + 1 < DNF:
                for c in loads[k + 1]:
                    c.start()
            if k >= NBUF:
                stores[k - NBUF].wait()
            transpose_chunk(bk, DW)
            stores[k].start()
        for k in range(max(0, DNF - NBUF), DNF):
            stores[k].wait()

        @pl.when(wid < n_extra)
        def _extra():
            run_chunk_sync(0, cat0(DNF), DW)

        @pl.when(wid == NW - 1)
        def _tail():
            run_chunk_sync(1, n_full * DW, tail)

        @pl.when(wid == 0)
        def _rem():
            r0 = (n_cat - rem) * D
            pltpu.sync_copy(rem_hbm, ob0.at[pl.ds(0, rem * D)])
            pltpu.sync_copy(
                ob0.at[pl.ds(0, rem * D)], out_hbm.at[pl.ds(r0, rem * D)]
            )

    return detile_kernel


@functools.lru_cache(maxsize=None)
def _make_gather(batch: int, n_col: int, n_cat: int):
    assert batch % (NW * 128) == 0
    b_per_w = batch // NW
    blk = 8 * b_per_w
    per_j = D * batch
    mesh = plsc.VectorSubcoreMesh(core_axis_name="c", subcore_axis_name="s")

    @functools.partial(
        pl.kernel,
        out_type=jax.ShapeDtypeStruct((n_col * per_j,), jnp.float32),
        mesh=mesh,
        compiler_params=pltpu.CompilerParams(
            use_tc_tiling_on_sc=False, needs_layout_passes=False
        ),
        scratch_types=[
            pltpu.VMEM((n_col, b_per_w), jnp.int32),
            pltpu.VMEM((NBUF, b_per_w, D), jnp.float32),
            pltpu.VMEM((NBUF, 2 * blk), jnp.float32),
            pltpu.SemaphoreType.DMA((NBUF,)),
            pltpu.SemaphoreType.DMA((NBUF,)),
        ],
    )
    def gather_kernel(idx_hbm, table_hbm, out_hbm, idx_v, rows_v, outt_v,
                      gsem, ssem):
        wid = lax.axis_index("s") * NC + lax.axis_index("c")
        base = wid * b_per_w
        pltpu.sync_copy(idx_hbm.at[:, pl.ds(base, b_per_w)], idx_v)
        gathers = [
            pltpu.make_async_copy(
                table_hbm.at[idx_v.at[j]],
                rows_v.at[j % NBUF],
                gsem.at[j % NBUF],
            )
            for j in range(n_col)
        ]
        stores = [
            [
                pltpu.make_async_copy(
                    outt_v.at[j % NBUF, pl.ds(rt * blk, blk)],
                    out_hbm.at[
                        pl.ds(j * per_j + rt * (8 * batch) + wid * blk, blk)
                    ],
                    ssem.at[j % NBUF],
                )
                for rt in range(2)
            ]
            for j in range(n_col)
        ]
        feat = lax.iota(jnp.int32, D)
        foff = (feat // 8) * blk + (feat % 8) * 128

        def transpose_block(buf):
            def outer(t, _):
                c0 = t * 128

                def inner(i, svec):
                    row = rows_v[buf, c0 + i, :]
                    plsc.store_scatter(outt_v.at[buf], [svec], row)
                    return svec + 1

                lax.fori_loop(0, 128, inner, foff + t * 1024, unroll=8)
                return 0

            lax.fori_loop(0, b_per_w // 128, outer, 0)

        gathers[0].start()
        for j in range(n_col):
            bj = j % NBUF
            gathers[j].wait()
            if j + 1 < n_col:
                gathers[j + 1].start()
            if j >= NBUF:
                for s in stores[j - NBUF]:
                    s.wait()
            transpose_block(bj)
            for s in stores[j]:
                s.start()
        for j in range(max(0, n_col - NBUF), n_col):
            for s in stores[j]:
                s.wait()

    return gather_kernel


def kernel(x, table):
    b, s = x.shape
    n_cat, d = table.shape
    xt = x.T.astype(jnp.int32)
    rem = n_cat % 128
    rem_rows = table[n_cat - rem:, :].reshape(-1)
    table_lin = _make_detile(n_cat)(table.T, rem_rows).reshape(n_cat, d)
    flat = _make_gather(b, s, n_cat)(xt, table_lin)
    out = flat.reshape(s, d // 8, b // 128, 8, 128)
    return out.transpose(2, 4, 0, 1, 3).reshape(b, s, d)


# speedup vs baseline: 2.8489x; 1.6690x over previous
import functools

import jax
import jax.numpy as jnp
from jax import lax
from jax.experimental import pallas as pl
from jax.experimental.pallas import tpu as pltpu
from jax.experimental.pallas import tpu_sc as plsc

D = 16
NC = 2
NS = 16
NW = NC * NS
NBUF = 2


DW = 1024
DNF = 30


@functools.lru_cache(maxsize=None)
def _make_detile(n_cat: int):
    n_full = n_cat // DW
    n_extra = n_full - DNF * NW
    rem = n_cat % 128
    tail = n_cat - n_full * DW - rem
    mesh = plsc.VectorSubcoreMesh(core_axis_name="c", subcore_axis_name="s")

    @functools.partial(
        pl.kernel,
        out_type=jax.ShapeDtypeStruct((n_cat * D,), jnp.float32),
        mesh=mesh,
        compiler_params=pltpu.CompilerParams(needs_layout_passes=False),
        scratch_types=[
            pltpu.VMEM((D * DW,), jnp.float32),
            pltpu.VMEM((D * DW,), jnp.float32),
            pltpu.VMEM((D * DW,), jnp.float32),
            pltpu.VMEM((D * DW,), jnp.float32),
            pltpu.SemaphoreType.DMA((NBUF,)),
            pltpu.SemaphoreType.DMA((NBUF,)),
        ],
    )
    def detile_kernel(tt_hbm, rem_hbm, out_hbm, in0, in1, ob0, ob1,
                      gsem, ssem):
        wid = lax.axis_index("s") * NC + lax.axis_index("c")
        lo = DNF * wid + jnp.minimum(wid, n_extra)
        feat = lax.iota(jnp.int32, D)
        inb = [in0, in1]
        outb = [ob0, ob1]

        def cat0(k):
            return pl.multiple_of((lo + k) * DW, 128)

        def transpose_chunk(buf, width):
            src, dst = inb[buf], outb[buf]

            @plsc.parallel_loop(0, width, 1, unroll=8, carry=feat * DW)
            def body(c, lvec):
                col = plsc.load_gather(src, [lvec])
                dst[pl.ds(c * D, D)] = col
                return lvec + 1

        def load_chunk(buf, c0, width):
            return [
                pltpu.make_async_copy(
                    tt_hbm.at[f, pl.ds(c0, width)],
                    inb[buf].at[pl.ds(f * DW, width)],
                    gsem.at[buf],
                )
                for f in range(D)
            ]

        def store_chunk(buf, c0, width):
            return pltpu.make_async_copy(
                outb[buf].at[pl.ds(0, width * D)],
                out_hbm.at[pl.ds(c0 * D, width * D)],
                ssem.at[buf],
            )

        def run_chunk_sync(buf, c0, width):
            for c in load_chunk(buf, c0, width):
                c.start()
            for c in load_chunk(buf, c0, width):
                c.wait()
            transpose_chunk(buf, width)
            store_chunk(buf, c0, width).start()
            store_chunk(buf, c0, width).wait()

        loads = [load_chunk(k % NBUF, cat0(k), DW) for k in range(DNF)]
        stores = [store_chunk(k % NBUF, cat0(k), DW) for k in range(DNF)]
        for c in loads[0]:
            c.start()
        for k in range(DNF):
            bk = k % NBUF
            for c in loads[k]:
                c.wait()
            if k + 1 < DNF:
                for c in loads[k + 1]:
                    c.start()
            if k >= NBUF:
                stores[k - NBUF].wait()
            transpose_chunk(bk, DW)
            stores[k].start()
        for k in range(max(0, DNF - NBUF), DNF):
            stores[k].wait()

        @pl.when(wid < n_extra)
        def _extra():
            run_chunk_sync(0, cat0(DNF), DW)

        @pl.when(wid == NW - 1)
        def _tail():
            run_chunk_sync(1, n_full * DW, tail)

        @pl.when(wid == 0)
        def _rem():
            r0 = (n_cat - rem) * D
            pltpu.sync_copy(rem_hbm, ob0.at[pl.ds(0, rem * D)])
            pltpu.sync_copy(
                ob0.at[pl.ds(0, rem * D)], out_hbm.at[pl.ds(r0, rem * D)]
            )

    return detile_kernel


@functools.lru_cache(maxsize=None)
def _make_gather(batch: int, n_col: int, n_cat: int):
    assert batch % (NW * 128) == 0
    b_per_w = batch // NW
    blk = 8 * b_per_w
    per_j = D * batch
    mesh = plsc.VectorSubcoreMesh(core_axis_name="c", subcore_axis_name="s")

    @functools.partial(
        pl.kernel,
        out_type=jax.ShapeDtypeStruct((n_col * per_j,), jnp.float32),
        mesh=mesh,
        compiler_params=pltpu.CompilerParams(
            use_tc_tiling_on_sc=False, needs_layout_passes=False
        ),
        scratch_types=[
            pltpu.VMEM((n_col, b_per_w), jnp.int32),
            pltpu.VMEM((NBUF, b_per_w, D), jnp.float32),
            pltpu.VMEM((NBUF, 2 * blk), jnp.float32),
            pltpu.SemaphoreType.DMA((NBUF,)),
            pltpu.SemaphoreType.DMA((NBUF,)),
        ],
    )
    def gather_kernel(idx_hbm, table_hbm, out_hbm, idx_v, rows_v, outt_v,
                      gsem, ssem):
        wid = lax.axis_index("s") * NC + lax.axis_index("c")
        base = wid * b_per_w
        pltpu.sync_copy(idx_hbm.at[:, pl.ds(base, b_per_w)], idx_v)
        gathers = [
            pltpu.make_async_copy(
                table_hbm.at[idx_v.at[j]],
                rows_v.at[j % NBUF],
                gsem.at[j % NBUF],
            )
            for j in range(n_col)
        ]
        stores = [
            [
                pltpu.make_async_copy(
                    outt_v.at[j % NBUF, pl.ds(rt * blk, blk)],
                    out_hbm.at[
                        pl.ds(j * per_j + rt * (8 * batch) + wid * blk, blk)
                    ],
                    ssem.at[j % NBUF],
                )
                for rt in range(2)
            ]
            for j in range(n_col)
        ]
        feat = lax.iota(jnp.int32, D)
        foff = (feat // 8) * blk + (feat % 8) * 128

        def transpose_block(buf):
            def outer(t, _):
                c0 = t * 128

                @plsc.parallel_loop(0, 128, 1, unroll=8, carry=foff + t * 1024)
                def inner(i, svec):
                    row = rows_v[buf, c0 + i, :]
                    plsc.store_scatter(outt_v.at[buf], [svec], row)
                    return svec + 1

                return 0

            lax.fori_loop(0, b_per_w // 128, outer, 0)

        gathers[0].start()
        for j in range(n_col):
            bj = j % NBUF
            gathers[j].wait()
            if j + 1 < n_col:
                gathers[j + 1].start()
            if j >= NBUF:
                for s in stores[j - NBUF]:
                    s.wait()
            transpose_block(bj)
            for s in stores[j]:
                s.start()
        for j in range(max(0, n_col - NBUF), n_col):
            for s in stores[j]:
                s.wait()

    return gather_kernel


def kernel(x, table):
    b, s = x.shape
    n_cat, d = table.shape
    xt = x.T.astype(jnp.int32)
    rem = n_cat % 128
    rem_rows = table[n_cat - rem:, :].reshape(-1)
    table_lin = _make_detile(n_cat)(table.T, rem_rows).reshape(n_cat, d)
    flat = _make_gather(b, s, n_cat)(xt, table_lin)
    out = flat.reshape(s, d // 8, b // 128, 8, 128)
    return out.transpose(2, 4, 0, 1, 3).reshape(b, s, d)
